# baseline (speedup 1.0000x reference)
import functools

import jax
import jax.numpy as jnp
from jax import lax
from jax.experimental import pallas as pl
from jax.experimental.pallas import tpu as pltpu
from jax.experimental.pallas import tpu_sc as plsc

_NC = 2
_NS = 16
_NW = _NC * _NS
_NBUF = 4


@functools.lru_cache(maxsize=None)
def _tc_flatten(V, D, blk):
    pack = 128 // D
    n_out = blk * D // 128
    assert blk % 128 == 0 and blk % pack == 0

    def body(in_ref, out_ref):
        x = in_ref[...]
        eye = jnp.eye(D, dtype=jnp.float32)
        y = lax.dot_general(
            x, eye, (((0,), (0,)), ((), ())),
            precision=lax.Precision.HIGHEST,
        )
        y3 = y.reshape(n_out, pack, D)
        out_ref[...] = jnp.concatenate(
            [y3[:, q, :] for q in range(pack)], axis=-1
        )

    return pl.pallas_call(
        body,
        grid=(pl.cdiv(V, blk),),
        in_specs=[pl.BlockSpec((D, blk), lambda g: (0, g))],
        out_specs=pl.BlockSpec((n_out, 128), lambda g: (g, 0)),
        out_shape=jax.ShapeDtypeStruct((V * D // 128, 128), jnp.float32),
    )


@functools.lru_cache(maxsize=None)
def _tc_untranspose(B, D, blk):
    pack = 128 // D
    n_in = B * D // 128
    jblk = blk * pack
    assert n_in % blk == 0

    def body(in_ref, out_ref):
        x = in_ref[...]
        y3 = jnp.stack(
            [x[:, q * D:(q + 1) * D] for q in range(pack)], axis=1
        )
        y = y3.reshape(jblk, D)
        eye = jnp.eye(D, dtype=jnp.float32)
        out_ref[...] = lax.dot_general(
            eye, y, (((0,), (1,)), ((), ())),
            precision=lax.Precision.HIGHEST,
        )

    return pl.pallas_call(
        body,
        grid=(n_in // blk,),
        in_specs=[pl.BlockSpec((blk, 128), lambda g: (g, 0))],
        out_specs=pl.BlockSpec((D, jblk), lambda g: (0, g)),
        out_shape=jax.ShapeDtypeStruct((D, B), jnp.float32),
    )


@functools.lru_cache(maxsize=None)
def _sc_gather(B, V, D, chunk):
    b_per_w = B // _NW
    n_chunks = b_per_w // chunk
    assert b_per_w * _NW == B and n_chunks * chunk == b_per_w
    assert n_chunks % _NBUF == 0 and n_chunks >= 2 * _NBUF

    mesh = plsc.VectorSubcoreMesh(core_axis_name="c", subcore_axis_name="s")

    @functools.partial(
        pl.kernel,
        mesh=mesh,
        out_type=jax.ShapeDtypeStruct((B, D), jnp.float32),
        compiler_params=pltpu.CompilerParams(use_tc_tiling_on_sc=False),
        scratch_types=(
            [pltpu.VMEM((b_per_w,), jnp.int32)]
            + [pltpu.VMEM((chunk, D), jnp.float32)] * _NBUF
            + [pltpu.SemaphoreType.DMA] * (2 * _NBUF)
        ),
    )
    def gather_kernel(data_hbm, table_hbm, out_hbm, idx_v, *bufs_and_sems):
        bufs = bufs_and_sems[:_NBUF]
        gs = bufs_and_sems[_NBUF:2 * _NBUF]
        ws = bufs_and_sems[2 * _NBUF:]

        wid = lax.axis_index("s") * _NC + lax.axis_index("c")
        base = wid * b_per_w
        pltpu.sync_copy(data_hbm.at[pl.ds(base, b_per_w)], idx_v)

        def start_gather(j, b):
            pltpu.async_copy(
                table_hbm.at[idx_v.at[pl.ds(j * chunk, chunk)]], bufs[b], gs[b]
            )

        start_gather(0, 0)
        start_gather(1, 1)

        def body(h, carry):
            for off in range(_NBUF):
                j = _NBUF * h + off
                pltpu.make_async_copy(
                    table_hbm.at[idx_v.at[pl.ds(0, chunk)]], bufs[off], gs[off]
                ).wait()
                pltpu.async_copy(
                    bufs[off], out_hbm.at[pl.ds(base + j * chunk, chunk)],
                    ws[off],
                )
                nb = (off + 2) % _NBUF

                if off < 2:
                    @pl.when(h > 0)
                    def _():
                        pltpu.make_async_copy(
                            bufs[nb], out_hbm.at[pl.ds(base, chunk)], ws[nb]
                        ).wait()

                    @pl.when(j + 2 < n_chunks)
                    def _():
                        start_gather(j + 2, nb)
                else:
                    pltpu.make_async_copy(
                        bufs[nb], out_hbm.at[pl.ds(base, chunk)], ws[nb]
                    ).wait()

                    @pl.when(j + 2 < n_chunks)
                    def _():
                        start_gather(j + 2, nb)

            return carry

        lax.fori_loop(0, n_chunks // _NBUF, body, 0)
        pltpu.make_async_copy(
            bufs[2], out_hbm.at[pl.ds(base, chunk)], ws[2]
        ).wait()
        pltpu.make_async_copy(
            bufs[3], out_hbm.at[pl.ds(base, chunk)], ws[3]
        ).wait()

    return gather_kernel


def kernel(data, batch_sizes, table):
    B = data.shape[0]
    V, D = table.shape
    tt = table.T
    flat = _tc_flatten(V, D, 8064)(tt)
    table_rm = jnp.reshape(flat, (V, D))
    rows = _sc_gather(B, V, D, 640)(data.astype(jnp.int32), table_rm)
    rows_flat = jnp.reshape(rows, (B * D // 128, 128))
    out_t = _tc_untranspose(B, D, 1600)(rows_flat)
    return (out_t.T, batch_sizes)


# speedup vs baseline: 1.5335x; 1.5335x over previous
import functools

import jax
import jax.numpy as jnp
from jax import lax
from jax.experimental import pallas as pl
from jax.experimental.pallas import tpu as pltpu
from jax.experimental.pallas import tpu_sc as plsc

_NC = 2
_NS = 16
_NW = _NC * _NS
_NBUF = 4


@functools.lru_cache(maxsize=None)
def _make_gather(B, V, D, chunk):
    b_per_w = B // _NW
    n_chunks = b_per_w // chunk
    assert b_per_w * _NW == B and n_chunks * chunk == b_per_w
    assert n_chunks % _NBUF == 0 and n_chunks >= 2 * _NBUF

    mesh = plsc.VectorSubcoreMesh(core_axis_name="c", subcore_axis_name="s")

    @functools.partial(
        pl.kernel,
        mesh=mesh,
        out_type=jax.ShapeDtypeStruct((B, D), jnp.float32),
        compiler_params=pltpu.CompilerParams(use_tc_tiling_on_sc=False),
        scratch_types=(
            [pltpu.VMEM((b_per_w,), jnp.int32)]
            + [pltpu.VMEM((chunk, D), jnp.float32)] * _NBUF
            + [pltpu.SemaphoreType.DMA] * (2 * _NBUF)
        ),
    )
    def gather_kernel(data_hbm, table_hbm, out_hbm, idx_v, *bufs_and_sems):
        bufs = bufs_and_sems[:_NBUF]
        gs = bufs_and_sems[_NBUF:2 * _NBUF]
        ws = bufs_and_sems[2 * _NBUF:]

        wid = lax.axis_index("s") * _NC + lax.axis_index("c")
        base = wid * b_per_w
        pltpu.sync_copy(data_hbm.at[pl.ds(base, b_per_w)], idx_v)

        def start_gather(j, b):
            pltpu.async_copy(
                table_hbm.at[idx_v.at[pl.ds(j * chunk, chunk)]], bufs[b], gs[b]
            )

        start_gather(0, 0)
        start_gather(1, 1)

        def body(h, carry):
            for off in range(_NBUF):
                j = _NBUF * h + off
                pltpu.make_async_copy(
                    table_hbm.at[idx_v.at[pl.ds(0, chunk)]], bufs[off], gs[off]
                ).wait()
                pltpu.async_copy(
                    bufs[off], out_hbm.at[pl.ds(base + j * chunk, chunk)],
                    ws[off],
                )
                nb = (off + 2) % _NBUF

                if off < 2:
                    @pl.when(h > 0)
                    def _():
                        pltpu.make_async_copy(
                            bufs[nb], out_hbm.at[pl.ds(base, chunk)], ws[nb]
                        ).wait()

                    @pl.when(j + 2 < n_chunks)
                    def _():
                        start_gather(j + 2, nb)
                else:
                    pltpu.make_async_copy(
                        bufs[nb], out_hbm.at[pl.ds(base, chunk)], ws[nb]
                    ).wait()

                    @pl.when(j + 2 < n_chunks)
                    def _():
                        start_gather(j + 2, nb)

            return carry

        lax.fori_loop(0, n_chunks // _NBUF, body, 0)
        pltpu.make_async_copy(
            bufs[2], out_hbm.at[pl.ds(base, chunk)], ws[2]
        ).wait()
        pltpu.make_async_copy(
            bufs[3], out_hbm.at[pl.ds(base, chunk)], ws[3]
        ).wait()

    return gather_kernel


def kernel(data, batch_sizes, table):
    B = data.shape[0]
    V, D = table.shape
    table_flat = jax.lax.optimization_barrier(jnp.reshape(table, (-1)))
    table_rm = jnp.reshape(table_flat, (V, D))
    embedded = _make_gather(B, V, D, 640)(data.astype(jnp.int32), table_rm)
    return (embedded, batch_sizes)


# speedup vs baseline: 1.7139x; 1.1177x over previous
import functools

import jax
import jax.numpy as jnp
from jax import lax
from jax.experimental import pallas as pl
from jax.experimental.pallas import tpu as pltpu
from jax.experimental.pallas import tpu_sc as plsc

_NC = 2
_NS = 16
_NW = _NC * _NS
_NBUF = 4


@functools.lru_cache(maxsize=None)
def _tc_flatten(V, D, blk):
    pack = 128 // D
    n_out = blk * D // 128
    assert blk % 128 == 0 and blk % pack == 0

    def body(in_ref, out_ref):
        x = in_ref[...]
        y = x.T
        y3 = y.reshape(n_out, pack, D)
        out_ref[...] = jnp.concatenate(
            [y3[:, q, :] for q in range(pack)], axis=-1
        )

    return pl.pallas_call(
        body,
        grid=(pl.cdiv(V, blk),),
        in_specs=[pl.BlockSpec((D, blk), lambda g: (0, g))],
        out_specs=pl.BlockSpec((n_out, 128), lambda g: (g, 0)),
        out_shape=jax.ShapeDtypeStruct((V * D // 128, 128), jnp.float32),
    )


@functools.lru_cache(maxsize=None)
def _tc_untranspose(B, D, blk):
    pack = 128 // D
    n_in = B * D // 128
    jblk = blk * pack
    assert n_in % blk == 0

    def body(in_ref, out_ref):
        x = in_ref[...]
        y3 = jnp.stack(
            [x[:, q * D:(q + 1) * D] for q in range(pack)], axis=1
        )
        y = y3.reshape(jblk, D)
        out_ref[...] = y.T

    return pl.pallas_call(
        body,
        grid=(n_in // blk,),
        in_specs=[pl.BlockSpec((blk, 128), lambda g: (g, 0))],
        out_specs=pl.BlockSpec((D, jblk), lambda g: (0, g)),
        out_shape=jax.ShapeDtypeStruct((D, B), jnp.float32),
    )


@functools.lru_cache(maxsize=None)
def _sc_gather(B, V, D, chunk):
    b_per_w = B // _NW
    n_chunks = b_per_w // chunk

    mesh = plsc.VectorSubcoreMesh(core_axis_name="c", subcore_axis_name="s")

    @functools.partial(
        pl.kernel,
        mesh=mesh,
        out_type=jax.ShapeDtypeStruct((B, D), jnp.float32),
        compiler_params=pltpu.CompilerParams(use_tc_tiling_on_sc=False),
        scratch_types=(
            [pltpu.VMEM((b_per_w,), jnp.int32)]
            + [pltpu.VMEM((chunk, D), jnp.float32)] * _NBUF
            + [pltpu.SemaphoreType.DMA] * (2 * _NBUF)
        ),
    )
    def gather_kernel(data_hbm, table_hbm, out_hbm, idx_v, *bufs_and_sems):
        bufs = bufs_and_sems[:_NBUF]
        gs = bufs_and_sems[_NBUF:2 * _NBUF]
        ws = bufs_and_sems[2 * _NBUF:]
        wid = lax.axis_index("s") * _NC + lax.axis_index("c")
        base = wid * b_per_w
        pltpu.sync_copy(data_hbm.at[pl.ds(base, b_per_w)], idx_v)

        def start_gather(j, b):
            pltpu.async_copy(
                table_hbm.at[idx_v.at[pl.ds(j * chunk, chunk)]], bufs[b], gs[b]
            )

        start_gather(0, 0)
        start_gather(1, 1)

        def body(h, carry):
            for off in range(_NBUF):
                j = _NBUF * h + off
                pltpu.make_async_copy(
                    table_hbm.at[idx_v.at[pl.ds(0, chunk)]], bufs[off], gs[off]
                ).wait()
                pltpu.async_copy(
                    bufs[off], out_hbm.at[pl.ds(base + j * chunk, chunk)],
                    ws[off],
                )
                nb = (off + 2) % _NBUF
                if off < 2:
                    @pl.when(h > 0)
                    def _():
                        pltpu.make_async_copy(
                            bufs[nb], out_hbm.at[pl.ds(base, chunk)], ws[nb]
                        ).wait()

                    @pl.when(j + 2 < n_chunks)
                    def _():
                        start_gather(j + 2, nb)
                else:
                    pltpu.make_async_copy(
                        bufs[nb], out_hbm.at[pl.ds(base, chunk)], ws[nb]
                    ).wait()

                    @pl.when(j + 2 < n_chunks)
                    def _():
                        start_gather(j + 2, nb)
            return carry

        lax.fori_loop(0, n_chunks // _NBUF, body, 0)
        pltpu.make_async_copy(
            bufs[2], out_hbm.at[pl.ds(base, chunk)], ws[2]
        ).wait()
        pltpu.make_async_copy(
            bufs[3], out_hbm.at[pl.ds(base, chunk)], ws[3]
        ).wait()

    return gather_kernel


def kernel(data, batch_sizes, table):
    B = data.shape[0]
    V, D = table.shape
    tt = table.T
    flat = _tc_flatten(V, D, 8064)(tt)
    table_rm = jnp.reshape(flat, (V, D))
    rows = _sc_gather(B, V, D, 640)(data.astype(jnp.int32), table_rm)
    return (rows, batch_sizes)
